# Initial kernel scaffold; baseline (speedup 1.0000x reference)
import jax
import jax.numpy as jnp
from jax.experimental import pallas as pl


def kernel(x, Wi, bi, emb, conn, Wq, Wk, Wv, ln_g, ln_b, Wo, bo):
    raise NotImplementedError("write your pallas kernel here")


# speedup vs baseline: 5.1921x; 5.1921x over previous
import functools

import jax
import jax.numpy as jnp
from jax.experimental import pallas as pl

TOPK_N = 5
_NEG = float("-inf")
_LANE = 128


def _dot(a, b, trans_b=False):
    dims = (((1,), (1,) if trans_b else (0,)), ((), ()))
    return jax.lax.dot_general(a, b, dims,
                               precision=jax.lax.Precision.DEFAULT,
                               preferred_element_type=jnp.float32)


def _dot_f32(a, b):
    return jax.lax.dot_general(a, b, (((1,), (0,)), ((), ())),
                               precision=jax.lax.Precision.HIGHEST,
                               preferred_element_type=jnp.float32)


def _rowsum(x):
    n = x.shape[-1]
    p = x[:, 0:_LANE]
    for c in range(1, n // _LANE):
        p = p + x[:, c * _LANE:(c + 1) * _LANE]
    g = p[:, 0:8]
    for k in range(1, 16):
        g = g + p[:, 8 * k:8 * (k + 1)]
    a = g[:, 0:4] + g[:, 4:8]
    b = a[:, 0:2] + a[:, 2:4]
    return b[:, 0:1] + b[:, 1:2]


def _rowmax_lanes(x):
    n = x.shape[-1]
    m = x[:, 0:_LANE]
    for c in range(1, n // _LANE):
        m = jnp.maximum(m, x[:, c * _LANE:(c + 1) * _LANE])
    return m


def _lanemax(m):
    return jnp.max(m, axis=-1, keepdims=True)


def _h0_body(x_ref, wi_ref, bi_ref, emb_ref, o_ref):
    o_ref[0] = _dot(x_ref[0], wi_ref[...]) + bi_ref[...] + emb_ref[...]


def _h0(x, Wi, bi, emb, blk):
    B, A, D_IN = x.shape
    H = Wi.shape[1]
    grid = (B, A // blk)
    return pl.pallas_call(
        _h0_body,
        grid=grid,
        in_specs=[
            pl.BlockSpec((1, blk, D_IN), lambda b, a: (b, a, 0)),
            pl.BlockSpec((D_IN, H), lambda b, a: (0, 0)),
            pl.BlockSpec((1, H), lambda b, a: (0, 0)),
            pl.BlockSpec((blk, H), lambda b, a: (a, 0)),
        ],
        out_specs=pl.BlockSpec((1, blk, H), lambda b, a: (b, a, 0)),
        out_shape=jax.ShapeDtypeStruct((B, A, H), jnp.float32),
    )(x, Wi, bi.reshape(1, H), emb)


def _qkv_body(h_ref, wq_ref, wk_ref, wv_ref, q_ref, k_ref, v_ref):
    h = h_ref[0]
    q_ref[0, 0] = _dot(h, wq_ref[0])
    k_ref[0, 0] = _dot(h, wk_ref[0])
    v_ref[0, 0] = _dot(h, wv_ref[0])


def _qkv(h, Wq, Wk, Wv, blk):
    B, A, H = h.shape
    NH = Wq.shape[0]
    grid = (NH, B, A // blk)
    qkv_shape = jax.ShapeDtypeStruct((NH, B, A, H), jnp.float32)
    w_spec = pl.BlockSpec((1, H, H), lambda hd, b, a: (hd, 0, 0))
    o_spec = pl.BlockSpec((1, 1, blk, H), lambda hd, b, a: (hd, b, a, 0))
    return pl.pallas_call(
        _qkv_body,
        grid=grid,
        in_specs=[
            pl.BlockSpec((1, blk, H), lambda hd, b, a: (b, a, 0)),
            w_spec, w_spec, w_spec,
        ],
        out_specs=[o_spec, o_spec, o_spec],
        out_shape=[qkv_shape, qkv_shape, qkv_shape],
    )(h, Wq, Wk, Wv)


def _attn_body(q_ref, k_ref, v_ref, conn_ref, g_ref, bb_ref, o_ref, *,
               scale, nheads):
    hd = pl.program_id(2)
    s = _dot(q_ref[0, 0], k_ref[0, 0], trans_b=True) / scale + conn_ref[...]
    work = s
    for _ in range(TOPK_N - 1):
        m = jnp.max(work, axis=-1, keepdims=True)
        work = jnp.where(work >= m, _NEG, work)
    thresh = jnp.max(work, axis=-1, keepdims=True)
    mask = jnp.where(s >= thresh, s, _NEG)
    m_vec = _rowmax_lanes(mask)
    m_row = _lanemax(m_vec)
    n = mask.shape[-1]
    e1_chunks = [jnp.exp(mask[:, c * _LANE:(c + 1) * _LANE] - m_vec)
                 for c in range(n // _LANE)]
    f = jnp.exp(m_vec - m_row)
    e_eff = jnp.concatenate([ec * f for ec in e1_chunks], axis=-1)
    unnorm = _dot_f32(e_eff, v_ref[0, 0])
    denom = _rowsum(e_eff)
    out = unnorm / denom
    mu = _rowsum(out) * (1.0 / out.shape[-1])
    var = _rowsum(jnp.square(out - mu)) * (1.0 / out.shape[-1])
    ln = (out - mu) / jnp.sqrt(var + 1e-5) * g_ref[0] + bb_ref[0]

    @pl.when(hd == 0)
    def _():
        o_ref[0] = ln

    @pl.when(hd > 0)
    def _():
        o_ref[0] = o_ref[0] + ln

    @pl.when(hd == nheads - 1)
    def _():
        o = o_ref[0] * (1.0 / nheads)
        o_ref[0] = o * jax.nn.sigmoid(o)


def _attn(Q, K, V, conn, ln_g, ln_b, blk, scale):
    NH, B, A, H = Q.shape
    grid = (B, A // blk, NH)
    body = functools.partial(_attn_body, scale=scale, nheads=NH)
    kv_spec = pl.BlockSpec((1, 1, A, H), lambda b, q, hd: (hd, b, 0, 0))
    return pl.pallas_call(
        body,
        grid=grid,
        in_specs=[
            pl.BlockSpec((1, 1, blk, H), lambda b, q, hd: (hd, b, q, 0)),
            kv_spec, kv_spec,
            pl.BlockSpec((blk, A), lambda b, q, hd: (q, 0)),
            pl.BlockSpec((1, 1, H), lambda b, q, hd: (hd, 0, 0)),
            pl.BlockSpec((1, 1, H), lambda b, q, hd: (hd, 0, 0)),
        ],
        out_specs=pl.BlockSpec((1, blk, H), lambda b, q, hd: (b, q, 0)),
        out_shape=jax.ShapeDtypeStruct((B, A, H), jnp.float32),
    )(Q, K, V, conn, ln_g.reshape(NH, 1, H), ln_b.reshape(NH, 1, H))


def _fin_body(h_ref, wo_ref, bo_ref, o_ref):
    o_ref[0] = _dot(h_ref[0], wo_ref[...]) + bo_ref[...]


def _fin(h, Wo, bo, blk):
    B, A, H = h.shape
    D_OUT = Wo.shape[1]
    grid = (B, A // blk)
    return pl.pallas_call(
        _fin_body,
        grid=grid,
        in_specs=[
            pl.BlockSpec((1, blk, H), lambda b, a: (b, a, 0)),
            pl.BlockSpec((H, D_OUT), lambda b, a: (0, 0)),
            pl.BlockSpec((1, D_OUT), lambda b, a: (0, 0)),
        ],
        out_specs=pl.BlockSpec((1, blk, D_OUT), lambda b, a: (b, a, 0)),
        out_shape=jax.ShapeDtypeStruct((B, A, D_OUT), jnp.float32),
    )(h, Wo, bo.reshape(1, D_OUT))


def kernel(x, Wi, bi, emb, conn, Wq, Wk, Wv, ln_g, ln_b, Wo, bo):
    B, A, D_IN = x.shape
    H = Wi.shape[1]
    steps = 3
    scale = float(H) ** 0.5
    blk = min(256, A)
    h = _h0(x, Wi, bi, emb, blk)
    for _ in range(steps):
        Q, K, V = _qkv(h, Wq, Wk, Wv, blk)
        h = _attn(Q, K, V, conn, ln_g, ln_b, blk, scale)
    return _fin(h, Wo, bo, blk)


# speedup vs baseline: 5.5881x; 1.0763x over previous
import functools

import jax
import jax.numpy as jnp
from jax.experimental import pallas as pl

TOPK_N = 5
_NEG = float("-inf")
_LANE = 128


def _dot(a, b, trans_b=False):
    dims = (((1,), (1,) if trans_b else (0,)), ((), ()))
    return jax.lax.dot_general(a, b, dims,
                               precision=jax.lax.Precision.DEFAULT,
                               preferred_element_type=jnp.float32)


def _dot_f32(a, b):
    return jax.lax.dot_general(a, b, (((1,), (0,)), ((), ())),
                               precision=jax.lax.Precision.HIGHEST,
                               preferred_element_type=jnp.float32)


def _rowsum(x):
    n = x.shape[-1]
    p = x[:, 0:_LANE]
    for c in range(1, n // _LANE):
        p = p + x[:, c * _LANE:(c + 1) * _LANE]
    g = p[:, 0:8]
    for k in range(1, 16):
        g = g + p[:, 8 * k:8 * (k + 1)]
    a = g[:, 0:4] + g[:, 4:8]
    b = a[:, 0:2] + a[:, 2:4]
    return b[:, 0:1] + b[:, 1:2]


def _rowmax_lanes(x):
    n = x.shape[-1]
    m = x[:, 0:_LANE]
    for c in range(1, n // _LANE):
        m = jnp.maximum(m, x[:, c * _LANE:(c + 1) * _LANE])
    return m


def _lanemax(m):
    return jnp.max(m, axis=-1, keepdims=True)


def _h0_body(x_ref, wi_ref, bi_ref, emb_ref, o_ref):
    o_ref[0] = _dot(x_ref[0], wi_ref[...]) + bi_ref[...] + emb_ref[...]


def _h0(x, Wi, bi, emb, blk):
    B, A, D_IN = x.shape
    H = Wi.shape[1]
    grid = (B, A // blk)
    return pl.pallas_call(
        _h0_body,
        grid=grid,
        in_specs=[
            pl.BlockSpec((1, blk, D_IN), lambda b, a: (b, a, 0)),
            pl.BlockSpec((D_IN, H), lambda b, a: (0, 0)),
            pl.BlockSpec((1, H), lambda b, a: (0, 0)),
            pl.BlockSpec((blk, H), lambda b, a: (a, 0)),
        ],
        out_specs=pl.BlockSpec((1, blk, H), lambda b, a: (b, a, 0)),
        out_shape=jax.ShapeDtypeStruct((B, A, H), jnp.float32),
    )(x, Wi, bi.reshape(1, H), emb)


def _qkv_body(h_ref, wq_ref, wk_ref, wv_ref, q_ref, k_ref, v_ref):
    h = h_ref[0]
    q_ref[0, 0] = _dot(h, wq_ref[0])
    k_ref[0, 0] = _dot(h, wk_ref[0])
    v_ref[0, 0] = _dot(h, wv_ref[0])


def _qkv(h, Wq, Wk, Wv, blk):
    B, A, H = h.shape
    NH = Wq.shape[0]
    grid = (NH, B, A // blk)
    qkv_shape = jax.ShapeDtypeStruct((NH, B, A, H), jnp.float32)
    w_spec = pl.BlockSpec((1, H, H), lambda hd, b, a: (hd, 0, 0))
    o_spec = pl.BlockSpec((1, 1, blk, H), lambda hd, b, a: (hd, b, a, 0))
    return pl.pallas_call(
        _qkv_body,
        grid=grid,
        in_specs=[
            pl.BlockSpec((1, blk, H), lambda hd, b, a: (b, a, 0)),
            w_spec, w_spec, w_spec,
        ],
        out_specs=[o_spec, o_spec, o_spec],
        out_shape=[qkv_shape, qkv_shape, qkv_shape],
    )(h, Wq, Wk, Wv)


def _attn_body(q_ref, k_ref, v_ref, conn_ref, g_ref, bb_ref, o_ref, *,
               scale, nheads):
    hd = pl.program_id(2)
    s = _dot(q_ref[0, 0], k_ref[0, 0], trans_b=True) / scale + conn_ref[...]
    work = s
    for _ in range(TOPK_N - 1):
        m = jnp.max(work, axis=-1, keepdims=True)
        work = jnp.where(work >= m, _NEG, work)
    thresh = jnp.max(work, axis=-1, keepdims=True)
    mask = jnp.where(s >= thresh, s, _NEG)
    m_vec = _rowmax_lanes(mask)
    m_row = _lanemax(m_vec)
    n = mask.shape[-1]
    e1_chunks = [jnp.exp(mask[:, c * _LANE:(c + 1) * _LANE] - m_vec)
                 for c in range(n // _LANE)]
    f = jnp.exp(m_vec - m_row)
    e_eff = jnp.concatenate([ec * f for ec in e1_chunks], axis=-1)
    unnorm = _dot_f32(e_eff, v_ref[0, 0])
    denom = _rowsum(e_eff)
    out = unnorm / denom
    mu = _rowsum(out) * (1.0 / out.shape[-1])
    var = _rowsum(jnp.square(out - mu)) * (1.0 / out.shape[-1])
    ln = (out - mu) / jnp.sqrt(var + 1e-5) * g_ref[0] + bb_ref[0]

    @pl.when(hd == 0)
    def _():
        o_ref[0] = ln

    @pl.when(hd > 0)
    def _():
        o_ref[0] = o_ref[0] + ln

    @pl.when(hd == nheads - 1)
    def _():
        o = o_ref[0] * (1.0 / nheads)
        o_ref[0] = o * jax.nn.sigmoid(o)


def _attn(Q, K, V, conn, ln_g, ln_b, blk, scale):
    NH, B, A, H = Q.shape
    grid = (B, A // blk, NH)
    body = functools.partial(_attn_body, scale=scale, nheads=NH)
    kv_spec = pl.BlockSpec((1, 1, A, H), lambda b, q, hd: (hd, b, 0, 0))
    return pl.pallas_call(
        body,
        grid=grid,
        in_specs=[
            pl.BlockSpec((1, 1, blk, H), lambda b, q, hd: (hd, b, q, 0)),
            kv_spec, kv_spec,
            pl.BlockSpec((blk, A), lambda b, q, hd: (q, 0)),
            pl.BlockSpec((1, 1, H), lambda b, q, hd: (hd, 0, 0)),
            pl.BlockSpec((1, 1, H), lambda b, q, hd: (hd, 0, 0)),
        ],
        out_specs=pl.BlockSpec((1, blk, H), lambda b, q, hd: (b, q, 0)),
        out_shape=jax.ShapeDtypeStruct((B, A, H), jnp.float32),
    )(Q, K, V, conn, ln_g.reshape(NH, 1, H), ln_b.reshape(NH, 1, H))


def _fin_body(h_ref, wo_ref, bo_ref, o_ref):
    o_ref[0] = _dot(h_ref[0], wo_ref[...]) + bo_ref[...]


def _fin(h, Wo, bo, blk):
    B, A, H = h.shape
    D_OUT = Wo.shape[1]
    grid = (B, A // blk)
    return pl.pallas_call(
        _fin_body,
        grid=grid,
        in_specs=[
            pl.BlockSpec((1, blk, H), lambda b, a: (b, a, 0)),
            pl.BlockSpec((H, D_OUT), lambda b, a: (0, 0)),
            pl.BlockSpec((1, D_OUT), lambda b, a: (0, 0)),
        ],
        out_specs=pl.BlockSpec((1, blk, D_OUT), lambda b, a: (b, a, 0)),
        out_shape=jax.ShapeDtypeStruct((B, A, D_OUT), jnp.float32),
    )(h, Wo, bo.reshape(1, D_OUT))


def kernel(x, Wi, bi, emb, conn, Wq, Wk, Wv, ln_g, ln_b, Wo, bo):
    B, A, D_IN = x.shape
    H = Wi.shape[1]
    steps = 3
    scale = float(H) ** 0.5
    blk = min(256, A)
    blk_attn = min(512, A)
    h = _h0(x, Wi, bi, emb, blk)
    for _ in range(steps):
        Q, K, V = _qkv(h, Wq, Wk, Wv, blk)
        h = _attn(Q, K, V, conn, ln_g, ln_b, blk_attn, scale)
    return _fin(h, Wo, bo, blk)


# speedup vs baseline: 5.8395x; 1.0450x over previous
import functools

import jax
import jax.numpy as jnp
from jax.experimental import pallas as pl

TOPK_N = 5
_NEG = float("-inf")
_LANE = 128


def _dot(a, b, trans_b=False):
    dims = (((1,), (1,) if trans_b else (0,)), ((), ()))
    return jax.lax.dot_general(a, b, dims,
                               precision=jax.lax.Precision.DEFAULT,
                               preferred_element_type=jnp.float32)


def _dot_f32(a, b):
    return jax.lax.dot_general(a, b, (((1,), (0,)), ((), ())),
                               precision=jax.lax.Precision.HIGHEST,
                               preferred_element_type=jnp.float32)


def _rowsum(x):
    n = x.shape[-1]
    p = x[:, 0:_LANE]
    for c in range(1, n // _LANE):
        p = p + x[:, c * _LANE:(c + 1) * _LANE]
    g = p[:, 0:8]
    for k in range(1, 16):
        g = g + p[:, 8 * k:8 * (k + 1)]
    a = g[:, 0:4] + g[:, 4:8]
    b = a[:, 0:2] + a[:, 2:4]
    return b[:, 0:1] + b[:, 1:2]


def _rowmax_lanes(x):
    n = x.shape[-1]
    m = x[:, 0:_LANE]
    for c in range(1, n // _LANE):
        m = jnp.maximum(m, x[:, c * _LANE:(c + 1) * _LANE])
    return m


def _lanemax(m):
    return jnp.max(m, axis=-1, keepdims=True)


def _h0_body(x_ref, wi_ref, bi_ref, emb_ref, o_ref):
    o_ref[0] = _dot(x_ref[0], wi_ref[...]) + bi_ref[...] + emb_ref[...]


def _h0(x, Wi, bi, emb, blk):
    B, A, D_IN = x.shape
    H = Wi.shape[1]
    grid = (B, A // blk)
    return pl.pallas_call(
        _h0_body,
        grid=grid,
        in_specs=[
            pl.BlockSpec((1, blk, D_IN), lambda b, a: (b, a, 0)),
            pl.BlockSpec((D_IN, H), lambda b, a: (0, 0)),
            pl.BlockSpec((1, H), lambda b, a: (0, 0)),
            pl.BlockSpec((blk, H), lambda b, a: (a, 0)),
        ],
        out_specs=pl.BlockSpec((1, blk, H), lambda b, a: (b, a, 0)),
        out_shape=jax.ShapeDtypeStruct((B, A, H), jnp.float32),
    )(x, Wi, bi.reshape(1, H), emb)


def _qkv_body(h_ref, wq_ref, wk_ref, wv_ref, q_ref, k_ref, v_ref):
    h = h_ref[0]
    q_ref[0, 0] = _dot(h, wq_ref[0])
    k_ref[0, 0] = _dot(h, wk_ref[0])
    v_ref[0, 0] = _dot(h, wv_ref[0])


def _qkv(h, Wq, Wk, Wv, blk):
    B, A, H = h.shape
    NH = Wq.shape[0]
    grid = (NH, B, A // blk)
    qkv_shape = jax.ShapeDtypeStruct((NH, B, A, H), jnp.float32)
    w_spec = pl.BlockSpec((1, H, H), lambda hd, b, a: (hd, 0, 0))
    o_spec = pl.BlockSpec((1, 1, blk, H), lambda hd, b, a: (hd, b, a, 0))
    return pl.pallas_call(
        _qkv_body,
        grid=grid,
        in_specs=[
            pl.BlockSpec((1, blk, H), lambda hd, b, a: (b, a, 0)),
            w_spec, w_spec, w_spec,
        ],
        out_specs=[o_spec, o_spec, o_spec],
        out_shape=[qkv_shape, qkv_shape, qkv_shape],
    )(h, Wq, Wk, Wv)


def _attn_body(q_ref, k_ref, v_ref, conn_ref, g_ref, bb_ref, o_ref, *,
               scale, nheads):
    hd = pl.program_id(2)
    s = _dot(q_ref[0, 0], k_ref[0, 0], trans_b=True) / scale + conn_ref[...]
    work = s
    for _ in range(TOPK_N - 1):
        m = jnp.max(work, axis=-1, keepdims=True)
        work = jnp.where(work >= m, _NEG, work)
    thresh = jnp.max(work, axis=-1, keepdims=True)
    mask = jnp.where(s >= thresh, s, _NEG)
    m_vec = _rowmax_lanes(mask)
    m_row = _lanemax(m_vec)
    n = mask.shape[-1]
    e1_chunks = [jnp.exp(mask[:, c * _LANE:(c + 1) * _LANE] - m_vec)
                 for c in range(n // _LANE)]
    f = jnp.exp(m_vec - m_row)
    e_eff = jnp.concatenate([ec * f for ec in e1_chunks], axis=-1)
    unnorm = _dot_f32(e_eff, v_ref[0, 0])
    denom = _rowsum(e_eff)
    out = unnorm / denom
    mu = _rowsum(out) * (1.0 / out.shape[-1])
    var = _rowsum(jnp.square(out - mu)) * (1.0 / out.shape[-1])
    ln = (out - mu) / jnp.sqrt(var + 1e-5) * g_ref[0] + bb_ref[0]

    @pl.when(hd == 0)
    def _():
        o_ref[0] = ln

    @pl.when(hd > 0)
    def _():
        o_ref[0] = o_ref[0] + ln

    @pl.when(hd == nheads - 1)
    def _():
        o = o_ref[0] * (1.0 / nheads)
        o_ref[0] = o * jax.nn.sigmoid(o)


def _attn(Q, K, V, conn, ln_g, ln_b, blk, scale):
    NH, B, A, H = Q.shape
    grid = (B, A // blk, NH)
    body = functools.partial(_attn_body, scale=scale, nheads=NH)
    kv_spec = pl.BlockSpec((1, 1, A, H), lambda b, q, hd: (hd, b, 0, 0))
    return pl.pallas_call(
        body,
        grid=grid,
        in_specs=[
            pl.BlockSpec((1, 1, blk, H), lambda b, q, hd: (hd, b, q, 0)),
            kv_spec, kv_spec,
            pl.BlockSpec((blk, A), lambda b, q, hd: (q, 0)),
            pl.BlockSpec((1, 1, H), lambda b, q, hd: (hd, 0, 0)),
            pl.BlockSpec((1, 1, H), lambda b, q, hd: (hd, 0, 0)),
        ],
        out_specs=pl.BlockSpec((1, blk, H), lambda b, q, hd: (b, q, 0)),
        out_shape=jax.ShapeDtypeStruct((B, A, H), jnp.float32),
    )(Q, K, V, conn, ln_g.reshape(NH, 1, H), ln_b.reshape(NH, 1, H))


def _fin_body(h_ref, wo_ref, bo_ref, o_ref):
    o_ref[0] = _dot(h_ref[0], wo_ref[...]) + bo_ref[...]


def _fin(h, Wo, bo, blk):
    B, A, H = h.shape
    D_OUT = Wo.shape[1]
    grid = (B, A // blk)
    return pl.pallas_call(
        _fin_body,
        grid=grid,
        in_specs=[
            pl.BlockSpec((1, blk, H), lambda b, a: (b, a, 0)),
            pl.BlockSpec((H, D_OUT), lambda b, a: (0, 0)),
            pl.BlockSpec((1, D_OUT), lambda b, a: (0, 0)),
        ],
        out_specs=pl.BlockSpec((1, blk, D_OUT), lambda b, a: (b, a, 0)),
        out_shape=jax.ShapeDtypeStruct((B, A, D_OUT), jnp.float32),
    )(h, Wo, bo.reshape(1, D_OUT))


def kernel(x, Wi, bi, emb, conn, Wq, Wk, Wv, ln_g, ln_b, Wo, bo):
    B, A, D_IN = x.shape
    H = Wi.shape[1]
    steps = 3
    scale = float(H) ** 0.5
    blk = min(512, A)
    blk_attn = min(1024, A)
    h = _h0(x, Wi, bi, emb, blk)
    for _ in range(steps):
        Q, K, V = _qkv(h, Wq, Wk, Wv, blk)
        h = _attn(Q, K, V, conn, ln_g, ln_b, blk_attn, scale)
    return _fin(h, Wo, bo, blk)


# speedup vs baseline: 5.9185x; 1.0135x over previous
import functools

import jax
import jax.numpy as jnp
from jax.experimental import pallas as pl

TOPK_N = 5
_NEG = float("-inf")
_LANE = 128


def _dot(a, b, trans_b=False):
    dims = (((1,), (1,) if trans_b else (0,)), ((), ()))
    return jax.lax.dot_general(a, b, dims,
                               precision=jax.lax.Precision.DEFAULT,
                               preferred_element_type=jnp.float32)


def _dot_f32(a, b):
    return jax.lax.dot_general(a, b, (((1,), (0,)), ((), ())),
                               precision=jax.lax.Precision.HIGHEST,
                               preferred_element_type=jnp.float32)


def _rowsum(x):
    n = x.shape[-1]
    p = x[:, 0:_LANE]
    for c in range(1, n // _LANE):
        p = p + x[:, c * _LANE:(c + 1) * _LANE]
    g = p[:, 0:8]
    for k in range(1, 16):
        g = g + p[:, 8 * k:8 * (k + 1)]
    a = g[:, 0:4] + g[:, 4:8]
    b = a[:, 0:2] + a[:, 2:4]
    return b[:, 0:1] + b[:, 1:2]


def _rowmax_lanes(x):
    n = x.shape[-1]
    m = x[:, 0:_LANE]
    for c in range(1, n // _LANE):
        m = jnp.maximum(m, x[:, c * _LANE:(c + 1) * _LANE])
    return m


def _lanemax(m):
    return jnp.max(m, axis=-1, keepdims=True)


def _h0_body(x_ref, wi_ref, bi_ref, emb_ref, o_ref):
    o_ref[0] = _dot(x_ref[0], wi_ref[...]) + bi_ref[...] + emb_ref[...]


def _h0(x, Wi, bi, emb, blk):
    B, A, D_IN = x.shape
    H = Wi.shape[1]
    grid = (B, A // blk)
    return pl.pallas_call(
        _h0_body,
        grid=grid,
        in_specs=[
            pl.BlockSpec((1, blk, D_IN), lambda b, a: (b, a, 0)),
            pl.BlockSpec((D_IN, H), lambda b, a: (0, 0)),
            pl.BlockSpec((1, H), lambda b, a: (0, 0)),
            pl.BlockSpec((blk, H), lambda b, a: (a, 0)),
        ],
        out_specs=pl.BlockSpec((1, blk, H), lambda b, a: (b, a, 0)),
        out_shape=jax.ShapeDtypeStruct((B, A, H), jnp.float32),
    )(x, Wi, bi.reshape(1, H), emb)


def _qkv_body(h_ref, wq_ref, wk_ref, wv_ref, q_ref, k_ref, v_ref):
    h = h_ref[0]
    q_ref[0, 0] = _dot(h, wq_ref[0])
    k_ref[0, 0] = _dot(h, wk_ref[0])
    v_ref[0, 0] = _dot(h, wv_ref[0])


def _qkv(h, Wq, Wk, Wv, blk):
    B, A, H = h.shape
    NH = Wq.shape[0]
    grid = (NH, B, A // blk)
    qkv_shape = jax.ShapeDtypeStruct((NH, B, A, H), jnp.float32)
    w_spec = pl.BlockSpec((1, H, H), lambda hd, b, a: (hd, 0, 0))
    o_spec = pl.BlockSpec((1, 1, blk, H), lambda hd, b, a: (hd, b, a, 0))
    return pl.pallas_call(
        _qkv_body,
        grid=grid,
        in_specs=[
            pl.BlockSpec((1, blk, H), lambda hd, b, a: (b, a, 0)),
            w_spec, w_spec, w_spec,
        ],
        out_specs=[o_spec, o_spec, o_spec],
        out_shape=[qkv_shape, qkv_shape, qkv_shape],
    )(h, Wq, Wk, Wv)


def _attn_body(q_ref, k_ref, v_ref, conn_ref, g_ref, bb_ref, o_ref, *,
               scale, nheads):
    hd = pl.program_id(2)
    s = _dot(q_ref[0, 0], k_ref[0, 0], trans_b=True) / scale + conn_ref[...]
    work = s
    for _ in range(TOPK_N - 1):
        m = jnp.max(work, axis=-1, keepdims=True)
        work = jnp.where(work >= m, _NEG, work)
    thresh = jnp.max(work, axis=-1, keepdims=True)
    mask = jnp.where(s >= thresh, s, _NEG)
    m_vec = _rowmax_lanes(mask)
    m_row = _lanemax(m_vec)
    n = mask.shape[-1]
    e1_chunks = [jnp.exp(mask[:, c * _LANE:(c + 1) * _LANE] - m_vec)
                 for c in range(n // _LANE)]
    f = jnp.exp(m_vec - m_row)
    e_eff = jnp.concatenate([ec * f for ec in e1_chunks], axis=-1)
    unnorm = _dot_f32(e_eff, v_ref[0, 0])
    denom = _rowsum(e_eff)
    out = unnorm / denom
    mu = _rowsum(out) * (1.0 / out.shape[-1])
    var = _rowsum(jnp.square(out - mu)) * (1.0 / out.shape[-1])
    ln = (out - mu) / jnp.sqrt(var + 1e-5) * g_ref[0] + bb_ref[0]

    @pl.when(hd == 0)
    def _():
        o_ref[0] = ln

    @pl.when(hd > 0)
    def _():
        o_ref[0] = o_ref[0] + ln

    @pl.when(hd == nheads - 1)
    def _():
        o = o_ref[0] * (1.0 / nheads)
        o_ref[0] = o * jax.nn.sigmoid(o)


def _attn(Q, K, V, conn, ln_g, ln_b, blk, scale):
    NH, B, A, H = Q.shape
    grid = (B, A // blk, NH)
    body = functools.partial(_attn_body, scale=scale, nheads=NH)
    kv_spec = pl.BlockSpec((1, 1, A, H), lambda b, q, hd: (hd, b, 0, 0))
    return pl.pallas_call(
        body,
        grid=grid,
        in_specs=[
            pl.BlockSpec((1, 1, blk, H), lambda b, q, hd: (hd, b, q, 0)),
            kv_spec, kv_spec,
            pl.BlockSpec((blk, A), lambda b, q, hd: (q, 0)),
            pl.BlockSpec((1, 1, H), lambda b, q, hd: (hd, 0, 0)),
            pl.BlockSpec((1, 1, H), lambda b, q, hd: (hd, 0, 0)),
        ],
        out_specs=pl.BlockSpec((1, blk, H), lambda b, q, hd: (b, q, 0)),
        out_shape=jax.ShapeDtypeStruct((B, A, H), jnp.float32),
    )(Q, K, V, conn, ln_g.reshape(NH, 1, H), ln_b.reshape(NH, 1, H))


def _fin_body(h_ref, wo_ref, bo_ref, o_ref):
    o_ref[0] = _dot(h_ref[0], wo_ref[...]) + bo_ref[...]


def _fin(h, Wo, bo, blk):
    B, A, H = h.shape
    D_OUT = Wo.shape[1]
    grid = (B, A // blk)
    return pl.pallas_call(
        _fin_body,
        grid=grid,
        in_specs=[
            pl.BlockSpec((1, blk, H), lambda b, a: (b, a, 0)),
            pl.BlockSpec((H, D_OUT), lambda b, a: (0, 0)),
            pl.BlockSpec((1, D_OUT), lambda b, a: (0, 0)),
        ],
        out_specs=pl.BlockSpec((1, blk, D_OUT), lambda b, a: (b, a, 0)),
        out_shape=jax.ShapeDtypeStruct((B, A, D_OUT), jnp.float32),
    )(h, Wo, bo.reshape(1, D_OUT))


def kernel(x, Wi, bi, emb, conn, Wq, Wk, Wv, ln_g, ln_b, Wo, bo):
    B, A, D_IN = x.shape
    H = Wi.shape[1]
    steps = 3
    scale = float(H) ** 0.5
    blk = min(1024, A)
    blk_attn = min(1024, A)
    h = _h0(x, Wi, bi, emb, blk)
    for _ in range(steps):
        Q, K, V = _qkv(h, Wq, Wk, Wv, blk)
        h = _attn(Q, K, V, conn, ln_g, ln_b, blk_attn, scale)
    return _fin(h, Wo, bo, blk)
